# baseline (speedup 1.0000x reference)
import functools

import jax
import jax.numpy as jnp
from jax import lax
from jax.experimental import pallas as pl
from jax.experimental.pallas import tpu as pltpu
from jax.experimental.pallas import tpu_sc as plsc

NUM_USERS = 1000000
NUM_ITEMS = 1000000
EMB = 16
B = 16384

_info = plsc.get_sparse_core_info()
NC = _info.num_cores
NS = _info.num_subcores
L = _info.num_lanes
NW = NC * NS
BPW = B // NW
G = 16

_mesh = plsc.VectorSubcoreMesh(core_axis_name="c", subcore_axis_name="s")


@functools.partial(
    pl.kernel,
    mesh=_mesh,
    out_type=jax.ShapeDtypeStruct((B,), jnp.float32),
    scratch_types=[
        pltpu.VMEM((BPW,), jnp.int32),
        pltpu.VMEM((BPW,), jnp.int32),
        pltpu.VMEM((2, 2, 2, 8, 128), jnp.float32),
        pltpu.VMEM((2, 2, 2, 8, 128), jnp.float32),
        pltpu.VMEM((BPW,), jnp.float32),
        pltpu.SemaphoreType.DMA,
        pltpu.SemaphoreType.DMA,
    ],
    compiler_params=pltpu.CompilerParams(
        needs_layout_passes=False, use_tc_tiling_on_sc=True),
)
def _sc_predict(uid_hbm, iid_hbm, ue_hbm, ie_hbm, out_hbm,
                uid_v, iid_v, ub_v, vb_v, p_v, s0, s1):
    wid = lax.axis_index("s") * NC + lax.axis_index("c")
    base = wid * BPW
    pltpu.sync_copy(uid_hbm.at[pl.ds(base, BPW)], uid_v)
    pltpu.sync_copy(iid_hbm.at[pl.ds(base, BPW)], iid_v)

    lane = lax.iota(jnp.int32, L)
    sems = [s0, s1]
    R = 2
    NG = BPW // G

    def fire(gi, r, sem):
        ids_u = uid_v[pl.ds(gi * G, G)]
        ids_i = iid_v[pl.ds(gi * G, G)]
        wus = (ids_u // G) * G
        wis = (ids_i // G) * G
        for g in range(G):
            wu = pl.multiple_of(wus[g], 16)
            wi = pl.multiple_of(wis[g], 16)
            s2q, off = g // 8, 16 * (g % 8)
            pltpu.async_copy(
                ue_hbm.at[:, :, pl.ds(wu, G)],
                ub_v.at[r, s2q, :, :, pl.ds(off, G)], sem)
            pltpu.async_copy(
                ie_hbm.at[:, :, pl.ds(wi, G)],
                vb_v.at[r, s2q, :, :, pl.ds(off, G)], sem)

    for r in range(R):
        fire(r, r, sems[r])

    def group(gi, carry):
        r = lax.rem(gi, R)
        for s in range(R):

            @pl.when(r == s)
            def _(s=s):
                for _ in range(4):
                    pltpu.make_async_copy(
                        ue_hbm.at[:, :, pl.ds(0, 128)],
                        ub_v.at[0, 0], sems[s]).wait()

        ids_u = uid_v[pl.ds(gi * G, G)]
        ids_i = iid_v[pl.ds(gi * G, G)]
        cu = lax.rem(ids_u, G) + (lax.rem(lane, 8)) * G
        ci = lax.rem(ids_i, G) + (lax.rem(lane, 8)) * G
        rv = jnp.full((L,), 0, jnp.int32) + r
        s2v = lane // 8
        acc = jnp.zeros((L,), jnp.float32)
        for j in range(EMB):
            jhi = jnp.full((L,), j // 8, jnp.int32)
            jlo = jnp.full((L,), j % 8, jnp.int32)
            uu = plsc.load_gather(ub_v, [rv, s2v, jhi, jlo, cu])
            vv = plsc.load_gather(vb_v, [rv, s2v, jhi, jlo, ci])
            acc = acc + uu * vv
        p_v[pl.ds(gi * G, G)] = acc

        @pl.when(gi + R < NG)
        def _():
            for s in range(R):

                @pl.when(r == s)
                def _(s=s):
                    fire(gi + R, s, sems[s])

        return carry

    lax.fori_loop(0, NG, group, 0)
    pltpu.sync_copy(p_v, out_hbm.at[pl.ds(base, BPW)])


def _tc_loss_body(pred_ref, y_ref, out_ref):
    d = pred_ref[...] - y_ref[...]
    out_ref[0, 0] = jnp.sum(d * d) * (1.0 / B)


_tc_loss = pl.pallas_call(
    _tc_loss_body,
    out_shape=jax.ShapeDtypeStruct((1, 1), jnp.float32),
    out_specs=pl.BlockSpec(memory_space=pltpu.SMEM),
)


def kernel(uid, iid, y, user_emb, item_emb):
    uid = uid.astype(jnp.int32)
    iid = iid.astype(jnp.int32)
    ue3 = user_emb.T.reshape(2, 8, NUM_USERS)
    ie3 = item_emb.T.reshape(2, 8, NUM_ITEMS)
    pred = _sc_predict(uid, iid, ue3, ie3)
    loss = _tc_loss(pred.reshape(128, 128), y.reshape(128, 128))[0, 0]
    return pred, loss


# speedup vs baseline: 1.3367x; 1.3367x over previous
import functools

import jax
import jax.numpy as jnp
from jax import lax
from jax.experimental import pallas as pl
from jax.experimental.pallas import tpu as pltpu
from jax.experimental.pallas import tpu_sc as plsc

NUM_USERS = 1000000
NUM_ITEMS = 1000000
EMB = 16
B = 16384

_info = plsc.get_sparse_core_info()
NC = _info.num_cores
NS = _info.num_subcores
L = _info.num_lanes
NW = NC * NS
BPW = B // NW
G = 16

_mesh = plsc.VectorSubcoreMesh(core_axis_name="c", subcore_axis_name="s")


@functools.partial(
    pl.kernel,
    mesh=_mesh,
    out_type=jax.ShapeDtypeStruct((B,), jnp.float32),
    scratch_types=[
        pltpu.VMEM((BPW + L,), jnp.int32),
        pltpu.VMEM((BPW + L,), jnp.int32),
        pltpu.VMEM((2, 8, 2, 8, 128), jnp.float32),
        pltpu.VMEM((2, 8, 2, 8, 128), jnp.float32),
        pltpu.VMEM((BPW + L,), jnp.float32),
        pltpu.SemaphoreType.DMA,
        pltpu.SemaphoreType.DMA,
    ],
    compiler_params=pltpu.CompilerParams(
        needs_layout_passes=False, use_tc_tiling_on_sc=True),
)
def _sc_predict(uid_hbm, iid_hbm, ue_hbm, ie_hbm, out_hbm,
                uid_v, iid_v, ub_v, vb_v, p_v, s0, s1):
    wid = lax.axis_index("s") * NC + lax.axis_index("c")
    base = wid * BPW
    pltpu.sync_copy(uid_hbm.at[pl.ds(base, BPW)], uid_v.at[pl.ds(0, BPW)])
    pltpu.sync_copy(iid_hbm.at[pl.ds(base, BPW)], iid_v.at[pl.ds(0, BPW)])
    uid_v[pl.ds(BPW, L)] = jnp.zeros((L,), jnp.int32)
    iid_v[pl.ds(BPW, L)] = jnp.zeros((L,), jnp.int32)

    lane = lax.iota(jnp.int32, L)
    sems = [s0, s1]
    H = 8
    NH = BPW // H

    def fire(hi, r, sem):
        ids_u = uid_v[pl.ds(hi * H, L)]
        ids_i = iid_v[pl.ds(hi * H, L)]
        wus = (ids_u // G) * G
        wis = (ids_i // G) * G
        for g in range(H):
            wu = pl.multiple_of(wus[g], 16)
            wi = pl.multiple_of(wis[g], 16)
            pltpu.async_copy(
                ue_hbm.at[:, :, pl.ds(wu, G)],
                ub_v.at[r, g, :, :, pl.ds(0, G)], sem)
            pltpu.async_copy(
                ie_hbm.at[:, :, pl.ds(wi, G)],
                vb_v.at[r, g, :, :, pl.ds(0, G)], sem)

    fire(0, 0, sems[0])
    fire(1, 1, sems[1])

    lane8 = lax.rem(lane, H)
    half_mask = lane < H

    def half(hi, carry):
        r = lax.rem(hi, 2)
        for s in range(2):

            @pl.when(r == s)
            def _(s=s):
                for _ in range(2):
                    pltpu.make_async_copy(
                        ue_hbm.at[:, :, pl.ds(0, 128)],
                        ub_v.at[0, 0], sems[s]).wait()

        ids_u = uid_v[pl.ds(hi * H, L)]
        ids_i = iid_v[pl.ds(hi * H, L)]
        cu = lax.rem(ids_u, G)
        ci = lax.rem(ids_i, G)
        rv = jnp.full((L,), 0, jnp.int32) + r
        acc = jnp.zeros((L,), jnp.float32)
        for j in range(EMB):
            jhi = jnp.full((L,), j // 8, jnp.int32)
            jlo = jnp.full((L,), j % 8, jnp.int32)
            uu = plsc.load_gather(ub_v, [rv, lane8, jhi, jlo, cu])
            vv = plsc.load_gather(vb_v, [rv, lane8, jhi, jlo, ci])
            acc = acc + uu * vv
        plsc.store_compressed(p_v.at[pl.ds(hi * H, L)], acc, mask=half_mask)

        @pl.when(hi + 2 < NH)
        def _():
            for s in range(2):

                @pl.when(r == s)
                def _(s=s):
                    fire(hi + 2, s, sems[s])

        return carry

    lax.fori_loop(0, NH, half, 0)
    pltpu.sync_copy(p_v.at[pl.ds(0, BPW)], out_hbm.at[pl.ds(base, BPW)])


def _tc_loss_body(pred_ref, y_ref, out_ref):
    d = pred_ref[...] - y_ref[...]
    out_ref[0, 0] = jnp.sum(d * d) * (1.0 / B)


_tc_loss = pl.pallas_call(
    _tc_loss_body,
    out_shape=jax.ShapeDtypeStruct((1, 1), jnp.float32),
    out_specs=pl.BlockSpec(memory_space=pltpu.SMEM),
)


def kernel(uid, iid, y, user_emb, item_emb):
    uid = uid.astype(jnp.int32)
    iid = iid.astype(jnp.int32)
    ue3 = user_emb.T.reshape(2, 8, NUM_USERS)
    ie3 = item_emb.T.reshape(2, 8, NUM_ITEMS)
    pred = _sc_predict(uid, iid, ue3, ie3)
    loss = _tc_loss(pred.reshape(128, 128), y.reshape(128, 128))[0, 0]
    return pred, loss
